# baseline (speedup 1.0000x reference)
import functools

import jax
import jax.numpy as jnp
from jax import lax
from jax.experimental import pallas as pl
from jax.experimental.pallas import tpu as pltpu
from jax.experimental.pallas import tpu_sc as plsc

_D = 64
_SCALE = 8.0
_NC = 2
_NS = 16
_NW = _NC * _NS
_LANE = 128


def _iota16():
    return lax.iota(jnp.int32, 16)


def _sc_transpose(table_t):
    V = table_t.shape[1]
    nb = -(-V // _LANE)
    per_w = -(-nb // _NW)
    trn = nb * (_LANE // 2)

    mesh = plsc.VectorSubcoreMesh(core_axis_name="c", subcore_axis_name="s")

    @functools.partial(
        pl.kernel,
        out_type=jax.ShapeDtypeStruct((trn, _LANE), jnp.float32),
        mesh=mesh,
        scratch_types=[
            *[pltpu.VMEM((_D, _LANE), jnp.float32) for _ in range(4)],
            *[pltpu.SemaphoreType.DMA for _ in range(4)],
        ],
        compiler_params=pltpu.CompilerParams(needs_layout_passes=False),
    )
    def k(tab_hbm, tr_hbm, sb0, sb1, db0, db1, gi0, gi1, go0, go1):
        sbs = (sb0, sb1)
        dbs = (db0, db1)
        gis = (gi0, gi1)
        gos = (go0, go1)
        wid = lax.axis_index("s") * _NC + lax.axis_index("c")
        blk0 = wid * per_w
        it16 = _iota16()
        z16 = jnp.zeros((16,), jnp.int32)

        def in_desc(k_, p):
            return pltpu.make_async_copy(
                tab_hbm.at[:, pl.ds((blk0 + k_) * _LANE, _LANE)],
                sbs[p], gis[p])

        def out_desc(k_, p):
            return pltpu.make_async_copy(
                dbs[p], tr_hbm.at[pl.ds((blk0 + k_) * 64, 64)], gos[p])

        def valid(k_):
            return (blk0 + k_ < nb) & (k_ < per_w)

        @pl.when(valid(0))
        def _():
            in_desc(0, 0).start()

        def step(k_, p):
            @pl.when(valid(k_))
            def _():
                @pl.when(valid(k_ + 1))
                def _():
                    in_desc(k_ + 1, 1 - p).start()
                in_desc(k_, p).wait()

                @pl.when(k_ >= 2)
                def _():
                    out_desc(k_ - 2, p).wait()

                def trans(j, c_):
                    c0 = z16 + 2 * j
                    c1 = z16 + (2 * j + 1)
                    for q in range(4):
                        rq = it16 + 16 * q
                        g0 = plsc.load_gather(sbs[p], [rq, c0])
                        g1 = plsc.load_gather(sbs[p], [rq, c1])
                        dbs[p][j, pl.ds(16 * q, 16)] = g0
                        dbs[p][j, pl.ds(64 + 16 * q, 16)] = g1
                    return c_

                lax.fori_loop(0, 64, trans, 0)
                out_desc(k_, p).start()

        def group(g, carry):
            step(2 * g, 0)
            step(2 * g + 1, 1)
            return carry

        lax.fori_loop(0, (per_w + 1) // 2, group, 0,
                      unroll=False)
        for k_ in (per_w - 2, per_w - 1):
            if k_ >= 0:
                @pl.when(valid(k_))
                def _():
                    out_desc(k_, k_ % 2).wait()

    return k(table_t)


def _sc_gather_scale(tr, x_t, n_i):
    C = x_t.shape[0]
    nt = n_i // _LANE
    pairs = C * nt
    per_w = pairs // _NW

    mesh = plsc.VectorSubcoreMesh(core_axis_name="c", subcore_axis_name="s")

    @functools.partial(
        pl.kernel,
        out_type=jax.ShapeDtypeStruct((C, _D, n_i), jnp.float32),
        mesh=mesh,
        scratch_types=[
            pltpu.VMEM((_LANE,), jnp.int32),
            *[pltpu.VMEM((_LANE,), jnp.int32) for _ in range(2)],
            *[pltpu.VMEM((_LANE,), jnp.int32) for _ in range(2)],
            *[pltpu.VMEM((_LANE, _LANE), jnp.float32) for _ in range(2)],
            *[pltpu.VMEM((_D, _LANE), jnp.float32) for _ in range(2)],
            *[pltpu.SemaphoreType.DMA for _ in range(4)],
        ],
        compiler_params=pltpu.CompilerParams(needs_layout_passes=False),
    )
    def k(tr_hbm, x_hbm, out_hbm, xv, iv0, iv1, pv0, pv1,
          rw0, rw1, ot0, ot1, gg0, gg1, go0, go1):
        ivs = (iv0, iv1)
        pvs = (pv0, pv1)
        rws = (rw0, rw1)
        ots = (ot0, ot1)
        ggs = (gg0, gg1)
        gos = (go0, go1)
        wid = lax.axis_index("s") * _NC + lax.axis_index("c")
        p0 = wid * per_w
        it16 = _iota16()

        def c_of(k_):
            return (p0 + k_) // nt

        def ti_of(k_):
            return (p0 + k_) - nt * c_of(k_)

        def load_idx(k_, p):
            pltpu.sync_copy(
                x_hbm.at[c_of(k_), pl.ds(ti_of(k_) * _LANE, _LANE)], xv)
            for m in range(8):
                sl = pl.ds(16 * m, 16)
                v = xv[sl]
                ivs[p][sl] = v >> 1
                pvs[p][sl] = (v & 1) * 64

        def gather_desc(k_, p):
            return pltpu.make_async_copy(tr_hbm.at[ivs[p]], rws[p], ggs[p])

        def out_desc(k_, p):
            return pltpu.make_async_copy(
                ots[p],
                out_hbm.at[c_of(k_), :, pl.ds(ti_of(k_) * _LANE, _LANE)],
                gos[p])

        load_idx(0, 0)
        gather_desc(0, 0).start()

        def step(k_, p):
            @pl.when(k_ + 1 < per_w)
            def _():
                load_idx(k_ + 1, 1 - p)
                gather_desc(k_ + 1, 1 - p).start()
            gather_desc(k_, p).wait()

            @pl.when(k_ >= 2)
            def _():
                out_desc(k_ - 2, p).wait()

            parv = [pvs[p][pl.ds(16 * m, 16)] for m in range(8)]

            def assemble(d, c_):
                for m in range(8):
                    colv = parv[m] + d
                    g = plsc.load_gather(rws[p], [it16 + 16 * m, colv])
                    ots[p][d, pl.ds(16 * m, 16)] = g * _SCALE
                return c_

            lax.fori_loop(0, _D, assemble, 0)
            out_desc(k_, p).start()

        def group(g, carry):
            step(2 * g, 0)
            step(2 * g + 1, 1)
            return carry

        lax.fori_loop(0, per_w // 2, group, 0)
        out_desc(per_w - 2, 0).wait()
        out_desc(per_w - 1, 1).wait()

    return k(tr, x_t)


def kernel(x, lut_weight):
    xi = x.astype(jnp.int32)
    if (x.ndim == 2 and lut_weight.shape[1] == _D
            and x.shape[0] % _LANE == 0
            and (x.size // _LANE) % (2 * _NW) == 0):
        tr = _sc_transpose(lut_weight.T)
        out_t = _sc_gather_scale(tr, xi.T, x.shape[0])
        return out_t.transpose(2, 0, 1)
    n = x.size
    gran = _LANE * 2 * _NW
    pad = (-n) % gran
    flat = xi.reshape(-1)
    if pad:
        flat = jnp.concatenate([flat, jnp.zeros((pad,), jnp.int32)])
    tr = _sc_transpose(lut_weight.T)
    out_t = _sc_gather_scale(tr, flat.reshape(1, -1), flat.shape[0])
    out = out_t[0].T[:n]
    return out.reshape(*x.shape, _D)
